# baseline (speedup 1.0000x reference)
import functools

import jax
import jax.numpy as jnp
from jax import lax
from jax.experimental import pallas as pl
from jax.experimental.pallas import tpu as pltpu
from jax.experimental.pallas import tpu_sc as plsc

D_MODEL = 1024
D_VOCAB = 100000
NUM_TOKENS = 4 * 4096

_TR_BLOCK = 512


def _transpose_body(w_ref, out_ref):
    out_ref[...] = w_ref[...].T


def _transpose(w):
    grid = (pl.cdiv(D_VOCAB, _TR_BLOCK),)
    return pl.pallas_call(
        _transpose_body,
        grid=grid,
        in_specs=[pl.BlockSpec((D_MODEL, _TR_BLOCK), lambda i: (0, i))],
        out_specs=pl.BlockSpec((_TR_BLOCK, D_MODEL), lambda i: (i, 0)),
        out_shape=jax.ShapeDtypeStruct((D_VOCAB, D_MODEL), jnp.float32),
    )(w)


_NC = 2
_NS = 16
_NW = _NC * _NS
_B_PER_W = NUM_TOKENS // _NW
_CG = 32


def _sc_gather(table, tokens_flat):
    mesh = plsc.VectorSubcoreMesh(core_axis_name="c", subcore_axis_name="s")

    @functools.partial(
        pl.kernel,
        mesh=mesh,
        out_type=jax.ShapeDtypeStruct((NUM_TOKENS, D_MODEL), jnp.float32),
        scratch_types=[
            pltpu.VMEM((_B_PER_W,), jnp.int32),
            pltpu.VMEM((_CG, D_MODEL), jnp.float32),
            pltpu.SemaphoreType.DMA,
        ],
    )
    def k(table_hbm, idx_hbm, out_hbm, idx_v, rows_v, sem):
        wid = lax.axis_index("s") * _NC + lax.axis_index("c")
        base = wid * _B_PER_W
        pltpu.sync_copy(idx_hbm.at[pl.ds(base, _B_PER_W)], idx_v)

        def body(g, carry):
            off = g * _CG
            pltpu.async_copy(
                table_hbm.at[idx_v.at[pl.ds(off, _CG)]], rows_v, sem
            ).wait()
            pltpu.sync_copy(rows_v, out_hbm.at[pl.ds(base + off, _CG)])
            return carry

        lax.fori_loop(0, _B_PER_W // _CG, body, 0, unroll=False)

    return k(table, tokens_flat)


def kernel(tokens, W_E):
    w_t = _transpose(W_E)
    flat = _sc_gather(w_t, tokens.reshape(NUM_TOKENS))
    return flat.reshape(tokens.shape[0], tokens.shape[1], D_MODEL)
